# baseline (speedup 1.0000x reference)
import jax
import jax.numpy as jnp
from jax import lax
from jax.experimental import pallas as pl
from jax.experimental.pallas import tpu as pltpu

_NUM_CLASSES = 1000
_ROWS = 16384
_BLOCK_ROWS = 1024
_SPLIT = 896
_STRIP = 128


def _one_hot_block(x_ref, main_ref, strip_ref):
    idx = x_ref[:, 0]
    iota_main = jax.lax.broadcasted_iota(
        jnp.int32, (_BLOCK_ROWS, _SPLIT), 1)
    main_ref[...] = (idx[:, None] == iota_main).astype(jnp.int32)
    iota_strip = jax.lax.broadcasted_iota(
        jnp.int32, (_BLOCK_ROWS, _STRIP), 1) + _SPLIT
    strip_ref[...] = (idx[:, None] == iota_strip).astype(jnp.int32)


def kernel(x):
    idx = x.astype(jnp.int32)
    main, strip = pl.pallas_call(
        _one_hot_block,
        grid=(_ROWS // _BLOCK_ROWS,),
        in_specs=[pl.BlockSpec((_BLOCK_ROWS, 1), lambda i: (i, 0))],
        out_specs=[
            pl.BlockSpec((_BLOCK_ROWS, _SPLIT), lambda i: (i, 0)),
            pl.BlockSpec((_BLOCK_ROWS, _STRIP), lambda i: (i, 0)),
        ],
        out_shape=[
            jax.ShapeDtypeStruct((_ROWS, _NUM_CLASSES), jnp.int32),
            jax.ShapeDtypeStruct((_ROWS, _STRIP), jnp.int32),
        ],
        compiler_params=pltpu.CompilerParams(
            dimension_semantics=("parallel",)),
    )(idx)
    return lax.dynamic_update_slice(
        main, strip[:, : _NUM_CLASSES - _SPLIT], (0, _SPLIT))


# speedup vs baseline: 1.1092x; 1.1092x over previous
import jax
import jax.numpy as jnp
from jax.experimental import pallas as pl
from jax.experimental.pallas import tpu as pltpu

_NUM_CLASSES = 1000
_ROWS = 16384
_BLOCK_ROWS = 1024
_BLOCK_COLS = 1024


def _one_hot_block(x_ref, o_ref):
    idx = x_ref[:, 0]
    iota = jax.lax.broadcasted_iota(
        jnp.int32, (_BLOCK_ROWS, _BLOCK_COLS), 1)
    o_ref[...] = (idx[:, None] == iota).astype(jnp.int32)


def kernel(x):
    idx = x.astype(jnp.int32)
    return pl.pallas_call(
        _one_hot_block,
        grid=(_ROWS // _BLOCK_ROWS,),
        in_specs=[pl.BlockSpec((_BLOCK_ROWS, 1), lambda i: (i, 0))],
        out_specs=pl.BlockSpec((_BLOCK_ROWS, _BLOCK_COLS), lambda i: (i, 0)),
        out_shape=jax.ShapeDtypeStruct((_ROWS, _NUM_CLASSES), jnp.int32),
        compiler_params=pltpu.CompilerParams(
            dimension_semantics=("parallel",)),
    )(idx)
